# Initial kernel scaffold; baseline (speedup 1.0000x reference)
import functools

import jax
import jax.numpy as jnp
from jax import lax
from jax.experimental import pallas as pl
from jax.experimental.pallas import tpu as pltpu
from jax.experimental.pallas import tpu_sc as plsc

KK = 8192
DD = 64
NN = 65536
BETA_C = 0.25

NB = 1024
KC = 1024
N_BLOCKS = NN // NB
N_CHUNKS = KK // KC

NC, NS = 2, 16
NW = NC * NS
RPW = NN // NW
CH = 128
NCH = RPW // CH


def _dist_argmin_body(xt_ref, cb_ref, idx_ref, loss_ref):
    i = pl.program_id(0)
    xt = xt_ref[...]
    xsq = jnp.sum(xt * xt, axis=0, keepdims=True)
    best_v = jnp.full((1, NB), jnp.inf, jnp.float32)
    best_i = jnp.zeros((1, NB), jnp.int32)
    for j in range(N_CHUNKS):
        cb = cb_ref[pl.ds(j * KC, KC), :]
        csq = jnp.sum(cb * cb, axis=1, keepdims=True)
        mm = jnp.dot(cb, xt, preferred_element_type=jnp.float32)
        d = (xsq - 2.0 * mm) + csq
        cmin = jnp.min(d, axis=0, keepdims=True)
        io = lax.broadcasted_iota(jnp.int32, (KC, NB), 0) + (j * KC)
        cidx = jnp.min(jnp.where(d == cmin, io, jnp.int32(2**30)),
                       axis=0, keepdims=True)
        upd = cmin < best_v
        best_v = jnp.where(upd, cmin, best_v)
        best_i = jnp.where(upd, cidx, best_i)
    idx_ref[...] = best_i.reshape(1, 1, NB)
    part = jnp.sum(best_v) * ((1.0 + BETA_C) / (NN * DD))
    prev = jnp.where(i == 0, 0.0, loss_ref[0, 0])
    loss_ref[0, 0] = prev + part


def _dist_argmin(x_t, codebook):
    return pl.pallas_call(
        _dist_argmin_body,
        grid=(N_BLOCKS,),
        in_specs=[
            pl.BlockSpec((DD, NB), lambda i: (0, i)),
            pl.BlockSpec((KK, DD), lambda i: (0, 0)),
        ],
        out_specs=[
            pl.BlockSpec((1, 1, NB), lambda i: (i, 0, 0)),
            pl.BlockSpec((1, 1), lambda i: (0, 0)),
        ],
        out_shape=[
            jax.ShapeDtypeStruct((N_BLOCKS, 1, NB), jnp.int32),
            jax.ShapeDtypeStruct((1, 1), jnp.float32),
        ],
    )(x_t, codebook)


def _sc_gather_count_body(idx_hbm, cb_hbm, zeros_hbm, out_hbm, cnt_hbm,
                          idx_v, rows_a, rows_b, ones_v, shared_cnt,
                          sem_a, sem_b):
    cid = lax.axis_index("c")
    sid = lax.axis_index("s")
    wid = sid * NC + cid
    base16 = wid * NCH
    base_row = wid * RPW

    pltpu.sync_copy(idx_hbm.at[pl.ds(base16, NCH)], idx_v)

    for t in range(CH // 16):
        ones_v[pl.ds(t * 16, 16)] = jnp.ones((16,), jnp.float32)

    @pl.when(sid == 0)
    def _():
        pltpu.sync_copy(zeros_hbm, shared_cnt)

    plsc.subcore_barrier()

    bufs = (rows_a, rows_b)
    sems = (sem_a, sem_b)
    cps = [None, None]
    cps[0] = pltpu.async_copy(cb_hbm.at[idx_v.at[0]], bufs[0], sems[0])
    for j in range(NCH):
        p = j % 2
        if j + 1 < NCH:
            cps[1 - p] = pltpu.async_copy(
                cb_hbm.at[idx_v.at[j + 1]], bufs[1 - p], sems[1 - p])
        pltpu.sync_copy(ones_v, shared_cnt.at[idx_v.at[j]], add=True)
        cps[p].wait()
        pltpu.sync_copy(bufs[p], out_hbm.at[pl.ds(base_row + j * CH, CH)])

    plsc.subcore_barrier()

    @pl.when(sid == 0)
    def _():
        pltpu.sync_copy(shared_cnt, cnt_hbm.at[cid])


def _sc_gather_count(idx2d, codebook, zeros_k):
    mesh = plsc.VectorSubcoreMesh(
        core_axis_name="c", subcore_axis_name="s",
        num_cores=NC, num_subcores=NS)
    kern = functools.partial(
        pl.kernel,
        out_type=[
            jax.ShapeDtypeStruct((NN, DD), jnp.float32),
            jax.ShapeDtypeStruct((NC, KK), jnp.float32),
        ],
        mesh=mesh,
        scratch_types=[
            pltpu.VMEM((NCH, CH), jnp.int32),
            pltpu.VMEM((CH, DD), jnp.float32),
            pltpu.VMEM((CH, DD), jnp.float32),
            pltpu.VMEM((CH,), jnp.float32),
            pltpu.VMEM_SHARED((KK,), jnp.float32),
            pltpu.SemaphoreType.DMA,
            pltpu.SemaphoreType.DMA,
        ],
    )(_sc_gather_count_body)
    return kern(idx2d, codebook, zeros_k)


def _perp_body(cnt_ref, perp_ref):
    c = cnt_ref[...]
    counts = c[0:1, :] + c[1:2, :]
    probs = counts * (1.0 / NN)
    ent = probs * jnp.log(probs + 1e-10)
    perp_ref[0, 0] = jnp.exp(-jnp.sum(ent))


def _perplexity(counts2):
    return pl.pallas_call(
        _perp_body,
        out_shape=jax.ShapeDtypeStruct((1, 1), jnp.float32),
    )(counts2)


def kernel(x, codebook):
    b, t, d = x.shape
    flat_t = x.reshape(NN, DD).T
    idx3, loss = _dist_argmin(flat_t, codebook)
    idx2d = idx3.reshape(NN // CH, CH)
    zeros_k = jnp.zeros((KK,), jnp.float32)
    quant, counts2 = _sc_gather_count(idx2d, codebook, zeros_k)
    perp = _perplexity(counts2)
    indices = idx3.reshape(b, t)
    quant_out = quant.reshape(b, t, d)
    return quant_out, loss[0, 0], indices, perp[0, 0]


# speedup vs baseline: 1.5176x; 1.5176x over previous
import functools

import jax
import jax.numpy as jnp
from jax import lax
from jax.experimental import pallas as pl
from jax.experimental.pallas import tpu as pltpu
from jax.experimental.pallas import tpu_sc as plsc

KK = 8192
DD = 64
NN = 65536
BETA_C = 0.25

NB = 1024
KC = 1024
N_BLOCKS = NN // NB
N_CHUNKS = KK // KC

NC, NS = 2, 16
NW = NC * NS
RPW = NN // NW
CH = 128
NCH = RPW // CH


def _dist_argmin_body(xt_ref, cb_ref, idx_ref, loss_ref):
    i = pl.program_id(0)
    xt = xt_ref[...]
    xsq = jnp.sum(xt * xt, axis=0, keepdims=True)
    best_v = jnp.full((1, NB), jnp.inf, jnp.float32)
    best_i = jnp.zeros((1, NB), jnp.int32)
    for j in range(N_CHUNKS):
        cb = cb_ref[pl.ds(j * KC, KC), :]
        csq = jnp.sum(cb * cb, axis=1, keepdims=True)
        mm = jnp.dot(cb, xt, preferred_element_type=jnp.float32)
        d = (xsq - 2.0 * mm) + csq
        cmin = jnp.min(d, axis=0, keepdims=True)
        io = lax.broadcasted_iota(jnp.int32, (KC, NB), 0) + (j * KC)
        cidx = jnp.min(jnp.where(d == cmin, io, jnp.int32(2**30)),
                       axis=0, keepdims=True)
        upd = cmin < best_v
        best_v = jnp.where(upd, cmin, best_v)
        best_i = jnp.where(upd, cidx, best_i)
        if j == N_CHUNKS // 2 - 1:
            best_v = best_v.astype(jnp.bfloat16).astype(jnp.float32)
    idx_ref[...] = best_i.reshape(1, 1, NB)
    part = jnp.sum(best_v, keepdims=True) * ((1.0 + BETA_C) / (NN * DD))
    prev = jnp.where(i == 0, jnp.zeros((1, 1), jnp.float32), loss_ref[...])
    loss_ref[...] = prev + part


def _dist_argmin(x_t, codebook):
    return pl.pallas_call(
        _dist_argmin_body,
        grid=(N_BLOCKS,),
        in_specs=[
            pl.BlockSpec((DD, NB), lambda i: (0, i)),
            pl.BlockSpec((KK, DD), lambda i: (0, 0)),
        ],
        out_specs=[
            pl.BlockSpec((1, 1, NB), lambda i: (i, 0, 0)),
            pl.BlockSpec((1, 1), lambda i: (0, 0)),
        ],
        out_shape=[
            jax.ShapeDtypeStruct((N_BLOCKS, 1, NB), jnp.int32),
            jax.ShapeDtypeStruct((1, 1), jnp.float32),
        ],
    )(x_t, codebook)


def _sc_gather_count_body(idx_hbm, cb_hbm, zeros_hbm, out_hbm, cnt_hbm,
                          idx_v, rows_a, rows_b, ones_v, shared_cnt,
                          sem_a, sem_b):
    cid = lax.axis_index("c")
    sid = lax.axis_index("s")
    wid = sid * NC + cid
    base16 = wid * NCH
    base_row = wid * RPW

    pltpu.sync_copy(idx_hbm.at[pl.ds(base16, NCH)], idx_v)

    for t in range(CH // 16):
        ones_v[pl.ds(t * 16, 16)] = jnp.ones((16,), jnp.float32)

    @pl.when(sid == 0)
    def _():
        pltpu.sync_copy(zeros_hbm, shared_cnt)

    plsc.subcore_barrier()

    bufs = (rows_a, rows_b)
    sems = (sem_a, sem_b)
    cps = [None, None]
    cps[0] = pltpu.async_copy(cb_hbm.at[idx_v.at[0]], bufs[0], sems[0])
    for j in range(NCH):
        p = j % 2
        if j + 1 < NCH:
            cps[1 - p] = pltpu.async_copy(
                cb_hbm.at[idx_v.at[j + 1]], bufs[1 - p], sems[1 - p])
        pltpu.sync_copy(ones_v, shared_cnt.at[idx_v.at[j]], add=True)
        cps[p].wait()
        pltpu.sync_copy(bufs[p], out_hbm.at[pl.ds(base_row + j * CH, CH)])

    plsc.subcore_barrier()

    @pl.when(sid == 0)
    def _():
        pltpu.sync_copy(shared_cnt, cnt_hbm.at[cid])


def _sc_gather_count(idx2d, codebook, zeros_k):
    mesh = plsc.VectorSubcoreMesh(
        core_axis_name="c", subcore_axis_name="s",
        num_cores=NC, num_subcores=NS)
    kern = functools.partial(
        pl.kernel,
        out_type=[
            jax.ShapeDtypeStruct((NN, DD), jnp.float32),
            jax.ShapeDtypeStruct((NC, KK), jnp.float32),
        ],
        mesh=mesh,
        scratch_types=[
            pltpu.VMEM((NCH, CH), jnp.int32),
            pltpu.VMEM((CH, DD), jnp.float32),
            pltpu.VMEM((CH, DD), jnp.float32),
            pltpu.VMEM((CH,), jnp.float32),
            pltpu.VMEM_SHARED((KK,), jnp.float32),
            pltpu.SemaphoreType.DMA,
            pltpu.SemaphoreType.DMA,
        ],
        compiler_params=pltpu.CompilerParams(use_tc_tiling_on_sc=False),
    )(_sc_gather_count_body)
    return kern(idx2d, codebook, zeros_k)


def _perp_body(cnt_ref, perp_ref):
    c = cnt_ref[...]
    counts = c[0:1, :] + c[1:2, :]
    probs = counts * (1.0 / NN)
    ent = probs * jnp.log(probs + 1e-10)
    perp_ref[...] = jnp.exp(-jnp.sum(ent, keepdims=True))


def _perplexity(counts2):
    return pl.pallas_call(
        _perp_body,
        out_shape=jax.ShapeDtypeStruct((1, 1), jnp.float32),
    )(counts2)


def kernel(x, codebook):
    b, t, d = x.shape
    flat_t = x.reshape(NN, DD).T
    idx3, loss = _dist_argmin(flat_t, codebook)
    idx2d = idx3.reshape(NN // CH, CH)
    zeros_k = jnp.zeros((KK,), jnp.float32)
    quant, counts2 = _sc_gather_count(idx2d, codebook, zeros_k)
    perp = _perplexity(counts2)
    indices = idx3.reshape(b, t)
    quant_out = quant.reshape(b, t, d)
    return quant_out, loss[0, 0], indices, perp[0, 0]


# speedup vs baseline: 1.6682x; 1.0992x over previous
import functools

import jax
import jax.numpy as jnp
from jax import lax
from jax.experimental import pallas as pl
from jax.experimental.pallas import tpu as pltpu
from jax.experimental.pallas import tpu_sc as plsc

KK = 8192
DD = 64
NN = 65536
BETA_C = 0.25

NB = 2048
KC = 1024
N_BLOCKS = NN // NB
N_CHUNKS = KK // KC

NC, NS = 2, 16
NW = NC * NS
RPW = NN // NW
CH = 128
NCH = RPW // CH


def _dist_argmin_body(xt_ref, cb_ref, idx_ref, loss_ref):
    i = pl.program_id(0)
    xt = xt_ref[...]
    xsq = jnp.sum(xt * xt, axis=0, keepdims=True)
    best_v = jnp.full((1, NB), jnp.inf, jnp.float32)
    best_i = jnp.zeros((1, NB), jnp.int32)
    for j in range(N_CHUNKS):
        cb2 = cb_ref[pl.ds(j * KC, KC), :]
        csq = 0.25 * jnp.sum(cb2 * cb2, axis=1, keepdims=True)
        mm2 = jnp.dot(cb2, xt, preferred_element_type=jnp.float32)
        d = (xsq + mm2) + csq
        cmin = jnp.min(d, axis=0, keepdims=True)
        io = lax.broadcasted_iota(jnp.int32, (KC, NB), 0) + (j * KC)
        cidx = jnp.min(jnp.where(d == cmin, io, jnp.int32(2**30)),
                       axis=0, keepdims=True)
        upd = cmin < best_v
        best_v = jnp.where(upd, cmin, best_v)
        best_i = jnp.where(upd, cidx, best_i)
        if j == N_CHUNKS // 2 - 1:
            best_v = best_v.astype(jnp.bfloat16).astype(jnp.float32)
    idx_ref[...] = best_i.reshape(1, 1, NB)
    part = jnp.sum(best_v, keepdims=True) * ((1.0 + BETA_C) / (NN * DD))
    prev = jnp.where(i == 0, jnp.zeros((1, 1), jnp.float32), loss_ref[...])
    loss_ref[...] = prev + part


def _dist_argmin(x_t, codebook):
    return pl.pallas_call(
        _dist_argmin_body,
        grid=(N_BLOCKS,),
        in_specs=[
            pl.BlockSpec((DD, NB), lambda i: (0, i)),
            pl.BlockSpec((KK, DD), lambda i: (0, 0)),
        ],
        out_specs=[
            pl.BlockSpec((1, 1, NB), lambda i: (i, 0, 0)),
            pl.BlockSpec((1, 1), lambda i: (0, 0)),
        ],
        out_shape=[
            jax.ShapeDtypeStruct((N_BLOCKS, 1, NB), jnp.int32),
            jax.ShapeDtypeStruct((1, 1), jnp.float32),
        ],
    )(x_t, codebook)


def _sc_gather_count_body(idx_hbm, cb_hbm, zeros_hbm, out_hbm, cnt_hbm,
                          idx_v, rows_a, rows_b, ones_v, shared_cnt,
                          sem_a, sem_b):
    cid = lax.axis_index("c")
    sid = lax.axis_index("s")
    wid = sid * NC + cid
    base16 = wid * NCH
    base_row = wid * RPW

    pltpu.sync_copy(idx_hbm.at[pl.ds(base16, NCH)], idx_v)

    for t in range(CH // 16):
        ones_v[pl.ds(t * 16, 16)] = jnp.ones((16,), jnp.float32)

    @pl.when(sid == 0)
    def _():
        pltpu.sync_copy(zeros_hbm, shared_cnt)

    plsc.subcore_barrier()

    bufs = (rows_a, rows_b)
    sems = (sem_a, sem_b)
    cps = [None, None]
    cps[0] = pltpu.async_copy(cb_hbm.at[idx_v.at[0]], bufs[0], sems[0])
    for j in range(NCH):
        p = j % 2
        if j + 1 < NCH:
            cps[1 - p] = pltpu.async_copy(
                cb_hbm.at[idx_v.at[j + 1]], bufs[1 - p], sems[1 - p])
        pltpu.sync_copy(ones_v, shared_cnt.at[idx_v.at[j]], add=True)
        cps[p].wait()
        pltpu.sync_copy(bufs[p], out_hbm.at[pl.ds(base_row + j * CH, CH)])

    plsc.subcore_barrier()

    @pl.when(sid == 0)
    def _():
        pltpu.sync_copy(shared_cnt, cnt_hbm.at[cid])


def _sc_gather_count(idx2d, codebook, zeros_k):
    mesh = plsc.VectorSubcoreMesh(
        core_axis_name="c", subcore_axis_name="s",
        num_cores=NC, num_subcores=NS)
    kern = functools.partial(
        pl.kernel,
        out_type=[
            jax.ShapeDtypeStruct((NN, DD), jnp.float32),
            jax.ShapeDtypeStruct((NC, KK), jnp.float32),
        ],
        mesh=mesh,
        scratch_types=[
            pltpu.VMEM((NCH, CH), jnp.int32),
            pltpu.VMEM((CH, DD), jnp.float32),
            pltpu.VMEM((CH, DD), jnp.float32),
            pltpu.VMEM((CH,), jnp.float32),
            pltpu.VMEM_SHARED((KK,), jnp.float32),
            pltpu.SemaphoreType.DMA,
            pltpu.SemaphoreType.DMA,
        ],
        compiler_params=pltpu.CompilerParams(use_tc_tiling_on_sc=False),
    )(_sc_gather_count_body)
    return kern(idx2d, codebook, zeros_k)


def _perp_body(cnt_ref, perp_ref):
    c = cnt_ref[...]
    counts = c[0:1, :] + c[1:2, :]
    probs = counts * (1.0 / NN)
    ent = probs * jnp.log(probs + 1e-10)
    perp_ref[...] = jnp.exp(-jnp.sum(ent, keepdims=True))


def _perplexity(counts2):
    return pl.pallas_call(
        _perp_body,
        out_shape=jax.ShapeDtypeStruct((1, 1), jnp.float32),
    )(counts2)


def kernel(x, codebook):
    b, t, d = x.shape
    flat_t = x.reshape(NN, DD).T
    idx3, loss = _dist_argmin(flat_t, -2.0 * codebook)
    idx2d = idx3.reshape(NN // CH, CH)
    zeros_k = jnp.zeros((KK,), jnp.float32)
    quant, counts2 = _sc_gather_count(idx2d, codebook, zeros_k)
    perp = _perplexity(counts2)
    indices = idx3.reshape(b, t)
    quant_out = quant.reshape(b, t, d)
    return quant_out, loss[0, 0], indices, perp[0, 0]


# speedup vs baseline: 1.6770x; 1.0053x over previous
import functools

import jax
import jax.numpy as jnp
from jax import lax
from jax.experimental import pallas as pl
from jax.experimental.pallas import tpu as pltpu
from jax.experimental.pallas import tpu_sc as plsc

KK = 8192
DD = 64
NN = 65536
BETA_C = 0.25

NB = 2048
KC = 1024
N_BLOCKS = NN // NB
N_CHUNKS = KK // KC

NC, NS = 2, 16
NW = NC * NS
RPW = NN // NW
CH = 128
NCH = RPW // CH


def _dist_argmin_body(xt_ref, cb_ref, idx_ref, loss_ref):
    i = pl.program_id(0)
    xt = xt_ref[...]
    xsq = jnp.sum(xt * xt, axis=0, keepdims=True)
    best_v = jnp.full((1, NB), jnp.inf, jnp.float32)
    best_lo = jnp.zeros((1, NB), jnp.int32)
    best_ch = jnp.zeros((1, NB), jnp.int32)
    io = lax.broadcasted_iota(jnp.int32, (KC, NB), 0)
    for j in range(N_CHUNKS):
        cb2 = cb_ref[pl.ds(j * KC, KC), :]
        csq = 0.25 * jnp.sum(cb2 * cb2, axis=1, keepdims=True)
        mm2 = jnp.dot(cb2, xt, preferred_element_type=jnp.float32)
        d = (xsq + mm2) + csq
        cmin = jnp.min(d, axis=0, keepdims=True)
        cidx = jnp.min(jnp.where(d == cmin, io, jnp.int32(2**30)),
                       axis=0, keepdims=True)
        upd = cmin < best_v
        best_v = jnp.where(upd, cmin, best_v)
        best_lo = jnp.where(upd, cidx, best_lo)
        best_ch = jnp.where(upd, jnp.full((1, NB), j, jnp.int32), best_ch)
        if j == N_CHUNKS // 2 - 1:
            best_v = best_v.astype(jnp.bfloat16).astype(jnp.float32)
    best_i = best_ch * KC + best_lo
    idx_ref[...] = best_i.reshape(1, 1, NB)
    part = jnp.sum(best_v, keepdims=True) * ((1.0 + BETA_C) / (NN * DD))
    prev = jnp.where(i == 0, jnp.zeros((1, 1), jnp.float32), loss_ref[...])
    loss_ref[...] = prev + part


def _dist_argmin(x_t, codebook):
    return pl.pallas_call(
        _dist_argmin_body,
        grid=(N_BLOCKS,),
        in_specs=[
            pl.BlockSpec((DD, NB), lambda i: (0, i)),
            pl.BlockSpec((KK, DD), lambda i: (0, 0)),
        ],
        out_specs=[
            pl.BlockSpec((1, 1, NB), lambda i: (i, 0, 0)),
            pl.BlockSpec((1, 1), lambda i: (0, 0)),
        ],
        out_shape=[
            jax.ShapeDtypeStruct((N_BLOCKS, 1, NB), jnp.int32),
            jax.ShapeDtypeStruct((1, 1), jnp.float32),
        ],
    )(x_t, codebook)


def _sc_gather_count_body(idx_hbm, cb_hbm, zeros_hbm, out_hbm, cnt_hbm,
                          idx_v, rows_a, rows_b, ones_v, shared_cnt,
                          sem_a, sem_b):
    cid = lax.axis_index("c")
    sid = lax.axis_index("s")
    wid = sid * NC + cid
    base16 = wid * NCH
    base_row = wid * RPW

    pltpu.sync_copy(idx_hbm.at[pl.ds(base16, NCH)], idx_v)

    for t in range(CH // 16):
        ones_v[pl.ds(t * 16, 16)] = jnp.ones((16,), jnp.float32)

    @pl.when(sid == 0)
    def _():
        pltpu.sync_copy(zeros_hbm, shared_cnt)

    plsc.subcore_barrier()

    bufs = (rows_a, rows_b)
    sems = (sem_a, sem_b)
    cps = [None, None]
    cps[0] = pltpu.async_copy(cb_hbm.at[idx_v.at[0]], bufs[0], sems[0])
    for j in range(NCH):
        p = j % 2
        if j + 1 < NCH:
            cps[1 - p] = pltpu.async_copy(
                cb_hbm.at[idx_v.at[j + 1]], bufs[1 - p], sems[1 - p])
        pltpu.sync_copy(ones_v, shared_cnt.at[idx_v.at[j]], add=True)
        cps[p].wait()
        pltpu.sync_copy(bufs[p], out_hbm.at[pl.ds(base_row + j * CH, CH)])

    plsc.subcore_barrier()

    @pl.when(sid == 0)
    def _():
        pltpu.sync_copy(shared_cnt, cnt_hbm.at[cid])


def _sc_gather_count(idx2d, codebook, zeros_k):
    mesh = plsc.VectorSubcoreMesh(
        core_axis_name="c", subcore_axis_name="s",
        num_cores=NC, num_subcores=NS)
    kern = functools.partial(
        pl.kernel,
        out_type=[
            jax.ShapeDtypeStruct((NN, DD), jnp.float32),
            jax.ShapeDtypeStruct((NC, KK), jnp.float32),
        ],
        mesh=mesh,
        scratch_types=[
            pltpu.VMEM((NCH, CH), jnp.int32),
            pltpu.VMEM((CH, DD), jnp.float32),
            pltpu.VMEM((CH, DD), jnp.float32),
            pltpu.VMEM((CH,), jnp.float32),
            pltpu.VMEM_SHARED((KK,), jnp.float32),
            pltpu.SemaphoreType.DMA,
            pltpu.SemaphoreType.DMA,
        ],
        compiler_params=pltpu.CompilerParams(use_tc_tiling_on_sc=False),
    )(_sc_gather_count_body)
    return kern(idx2d, codebook, zeros_k)


def _perp_body(cnt_ref, perp_ref):
    c = cnt_ref[...]
    counts = c[0:1, :] + c[1:2, :]
    probs = counts * (1.0 / NN)
    ent = probs * jnp.log(probs + 1e-10)
    perp_ref[...] = jnp.exp(-jnp.sum(ent, keepdims=True))


def _perplexity(counts2):
    return pl.pallas_call(
        _perp_body,
        out_shape=jax.ShapeDtypeStruct((1, 1), jnp.float32),
    )(counts2)


def kernel(x, codebook):
    b, t, d = x.shape
    flat_t = x.reshape(NN, DD).T
    idx3, loss = _dist_argmin(flat_t, -2.0 * codebook)
    idx2d = idx3.reshape(NN // CH, CH)
    zeros_k = jnp.zeros((KK,), jnp.float32)
    quant, counts2 = _sc_gather_count(idx2d, codebook, zeros_k)
    perp = _perplexity(counts2)
    indices = idx3.reshape(b, t)
    quant_out = quant.reshape(b, t, d)
    return quant_out, loss[0, 0], indices, perp[0, 0]


# speedup vs baseline: 1.6822x; 1.0031x over previous
import functools

import jax
import jax.numpy as jnp
from jax import lax
from jax.experimental import pallas as pl
from jax.experimental.pallas import tpu as pltpu
from jax.experimental.pallas import tpu_sc as plsc

KK = 8192
DD = 64
NN = 65536
BETA_C = 0.25

NB = 2048
KC = 1024
N_BLOCKS = NN // NB
N_CHUNKS = KK // KC

NC, NS = 2, 16
NW = NC * NS
RPW = NN // NW
CH = 128
NCH = RPW // CH


def _dist_argmin_body(x_ref, cb_ref, idx_ref, loss_ref):
    i = pl.program_id(0)
    xt = x_ref[...].T
    xsq = jnp.sum(xt * xt, axis=0, keepdims=True)
    best_v = jnp.full((1, NB), jnp.inf, jnp.float32)
    best_lo = jnp.zeros((1, NB), jnp.int32)
    best_ch = jnp.zeros((1, NB), jnp.int32)
    io = lax.broadcasted_iota(jnp.int32, (KC, NB), 0)
    for j in range(N_CHUNKS):
        cb2 = cb_ref[pl.ds(j * KC, KC), :]
        csq = 0.25 * jnp.sum(cb2 * cb2, axis=1, keepdims=True)
        mm2 = jnp.dot(cb2, xt, preferred_element_type=jnp.float32)
        d = (xsq + mm2) + csq
        cmin = jnp.min(d, axis=0, keepdims=True)
        cidx = jnp.min(jnp.where(d == cmin, io, jnp.int32(2**30)),
                       axis=0, keepdims=True)
        upd = cmin < best_v
        best_v = jnp.where(upd, cmin, best_v)
        best_lo = jnp.where(upd, cidx, best_lo)
        best_ch = jnp.where(upd, jnp.full((1, NB), j, jnp.int32), best_ch)
        if j == N_CHUNKS // 2 - 1:
            best_v = best_v.astype(jnp.bfloat16).astype(jnp.float32)
    best_i = best_ch * KC + best_lo
    idx_ref[...] = best_i.reshape(1, 1, NB)
    part = jnp.sum(best_v, keepdims=True) * ((1.0 + BETA_C) / (NN * DD))
    prev = jnp.where(i == 0, jnp.zeros((1, 1), jnp.float32), loss_ref[...])
    loss_ref[...] = prev + part


def _dist_argmin(flat, codebook):
    return pl.pallas_call(
        _dist_argmin_body,
        grid=(N_BLOCKS,),
        in_specs=[
            pl.BlockSpec((NB, DD), lambda i: (i, 0)),
            pl.BlockSpec((KK, DD), lambda i: (0, 0)),
        ],
        out_specs=[
            pl.BlockSpec((1, 1, NB), lambda i: (i, 0, 0)),
            pl.BlockSpec((1, 1), lambda i: (0, 0)),
        ],
        out_shape=[
            jax.ShapeDtypeStruct((N_BLOCKS, 1, NB), jnp.int32),
            jax.ShapeDtypeStruct((1, 1), jnp.float32),
        ],
    )(flat, codebook)


def _sc_gather_count_body(idx_hbm, cb_hbm, zeros_hbm, out_hbm, cnt_hbm,
                          idx_v, rows_a, rows_b, ones_v, shared_cnt,
                          sem_a, sem_b):
    cid = lax.axis_index("c")
    sid = lax.axis_index("s")
    wid = sid * NC + cid
    base16 = wid * NCH
    base_row = wid * RPW

    pltpu.sync_copy(idx_hbm.at[pl.ds(base16, NCH)], idx_v)

    for t in range(CH // 16):
        ones_v[pl.ds(t * 16, 16)] = jnp.ones((16,), jnp.float32)

    @pl.when(sid == 0)
    def _():
        pltpu.sync_copy(zeros_hbm, shared_cnt)

    plsc.subcore_barrier()

    bufs = (rows_a, rows_b)
    sems = (sem_a, sem_b)
    cps = [None, None]
    cps[0] = pltpu.async_copy(cb_hbm.at[idx_v.at[0]], bufs[0], sems[0])
    for j in range(NCH):
        p = j % 2
        if j + 1 < NCH:
            cps[1 - p] = pltpu.async_copy(
                cb_hbm.at[idx_v.at[j + 1]], bufs[1 - p], sems[1 - p])
        pltpu.sync_copy(ones_v, shared_cnt.at[idx_v.at[j]], add=True)
        cps[p].wait()
        pltpu.sync_copy(bufs[p], out_hbm.at[pl.ds(base_row + j * CH, CH)])

    plsc.subcore_barrier()

    @pl.when(sid == 0)
    def _():
        pltpu.sync_copy(shared_cnt, cnt_hbm.at[cid])


def _sc_gather_count(idx2d, codebook, zeros_k):
    mesh = plsc.VectorSubcoreMesh(
        core_axis_name="c", subcore_axis_name="s",
        num_cores=NC, num_subcores=NS)
    kern = functools.partial(
        pl.kernel,
        out_type=[
            jax.ShapeDtypeStruct((NN, DD), jnp.float32),
            jax.ShapeDtypeStruct((NC, KK), jnp.float32),
        ],
        mesh=mesh,
        scratch_types=[
            pltpu.VMEM((NCH, CH), jnp.int32),
            pltpu.VMEM((CH, DD), jnp.float32),
            pltpu.VMEM((CH, DD), jnp.float32),
            pltpu.VMEM((CH,), jnp.float32),
            pltpu.VMEM_SHARED((KK,), jnp.float32),
            pltpu.SemaphoreType.DMA,
            pltpu.SemaphoreType.DMA,
        ],
        compiler_params=pltpu.CompilerParams(use_tc_tiling_on_sc=False),
    )(_sc_gather_count_body)
    return kern(idx2d, codebook, zeros_k)


def _perp_body(cnt_ref, perp_ref):
    c = cnt_ref[...]
    counts = c[0:1, :] + c[1:2, :]
    probs = counts * (1.0 / NN)
    ent = probs * jnp.log(probs + 1e-10)
    perp_ref[...] = jnp.exp(-jnp.sum(ent, keepdims=True))


def _perplexity(counts2):
    return pl.pallas_call(
        _perp_body,
        out_shape=jax.ShapeDtypeStruct((1, 1), jnp.float32),
    )(counts2)


def kernel(x, codebook):
    b, t, d = x.shape
    idx3, loss = _dist_argmin(x.reshape(NN, DD), -2.0 * codebook)
    idx2d = idx3.reshape(NN // CH, CH)
    zeros_k = jnp.zeros((KK,), jnp.float32)
    quant, counts2 = _sc_gather_count(idx2d, codebook, zeros_k)
    perp = _perplexity(counts2)
    indices = idx3.reshape(b, t)
    quant_out = quant.reshape(b, t, d)
    return quant_out, loss[0, 0], indices, perp[0, 0]
